# Initial kernel scaffold; baseline (speedup 1.0000x reference)
import functools

import jax
import jax.numpy as jnp
from jax import lax
from jax.experimental import pallas as pl
from jax.experimental.pallas import tpu as pltpu
from jax.experimental.pallas import tpu_sc as plsc

N_NODES = 10000
N_EDGES = 320000
D_N = 128
D_E = 16
D_CAT = 144

NC = 2
NS = 16
NW = NC * NS
E_PER_W = N_EDGES // NW
ROWS_PER_TILE = N_NODES // NS

GC = 400
N_GCHUNK = E_PER_W // GC
SCC = 1000
N_SCHUNK = E_PER_W // SCC

BE = 6400
BN = 2000

_mesh = plsc.VectorSubcoreMesh(core_axis_name="c", subcore_axis_name="s")


@functools.partial(
    pl.kernel,
    out_type=jax.ShapeDtypeStruct((N_EDGES, D_CAT), jnp.float32),
    mesh=_mesh,
    scratch_types=[
        pltpu.VMEM((GC,), jnp.int32),
        pltpu.VMEM((GC,), jnp.int32),
        pltpu.VMEM((GC, D_CAT), jnp.float32),
        pltpu.SemaphoreType.DMA,
    ],
)
def _gather_kernel(xw_hbm, src_hbm, dst_hbm, out_hbm, sidx, didx, rows, sem):
    c = lax.axis_index("c")
    s = lax.axis_index("s")
    base = (s * NC + c) * E_PER_W

    def body(k, carry):
        off = base + k * GC
        pltpu.sync_copy(src_hbm.at[pl.ds(off, GC)], sidx)
        pltpu.sync_copy(dst_hbm.at[pl.ds(off, GC)], didx)
        pltpu.async_copy(xw_hbm.at[sidx], rows, sem).wait()
        pltpu.async_copy(xw_hbm.at[didx], rows, sem, add=True).wait()
        pltpu.sync_copy(rows, out_hbm.at[pl.ds(off, GC)])
        return carry

    lax.fori_loop(0, N_GCHUNK, body, 0)


@functools.partial(
    pl.kernel,
    out_type=jax.ShapeDtypeStruct((NC, N_NODES, D_E), jnp.float32),
    mesh=_mesh,
    scratch_types=[
        pltpu.VMEM((SCC,), jnp.int32),
        pltpu.VMEM((SCC,), jnp.int32),
        pltpu.VMEM((SCC, D_E), jnp.float32),
        pltpu.VMEM((ROWS_PER_TILE, D_E), jnp.float32),
        pltpu.VMEM_SHARED((N_NODES, D_E), jnp.float32),
        pltpu.SemaphoreType.DMA,
    ],
)
def _scatter_kernel(eo_hbm, src_hbm, dst_hbm, out_hbm,
                    sidx, didx, vals, zbuf, aggsh, sem):
    c = lax.axis_index("c")
    s = lax.axis_index("s")
    wid = s * NC + c

    def zb(i, carry):
        zbuf[i, :] = jnp.zeros((D_E,), jnp.float32)
        return carry

    lax.fori_loop(0, ROWS_PER_TILE, zb, 0)
    pltpu.sync_copy(zbuf, aggsh.at[pl.ds(s * ROWS_PER_TILE, ROWS_PER_TILE)])
    plsc.subcore_barrier()

    def body(k, carry):
        off = wid * E_PER_W + k * SCC
        pltpu.sync_copy(src_hbm.at[pl.ds(off, SCC)], sidx)
        pltpu.sync_copy(dst_hbm.at[pl.ds(off, SCC)], didx)
        pltpu.sync_copy(eo_hbm.at[pl.ds(off, SCC)], vals)
        pltpu.sync_copy(vals, aggsh.at[sidx], add=True)
        pltpu.sync_copy(vals, aggsh.at[didx], add=True)
        return carry

    lax.fori_loop(0, N_SCHUNK, body, 0)
    plsc.subcore_barrier()
    pltpu.sync_copy(
        aggsh.at[pl.ds(s * ROWS_PER_TILE, ROWS_PER_TILE)],
        out_hbm.at[c, pl.ds(s * ROWS_PER_TILE, ROWS_PER_TILE)],
    )


def _xw_body(x_ref, w_ref, o_ref):
    o_ref[...] = jnp.dot(x_ref[...], w_ref[...],
                         preferred_element_type=jnp.float32)


def _edge_body(g_ref, ea_ref, wtop_ref, be1a_ref, we1b_ref, be1b_ref,
               we2a_ref, be2a_ref, we2b_ref, be2b_ref, eps_ref, o_ref):
    ea = ea_ref[...]
    h1 = g_ref[...] + jnp.dot(ea, wtop_ref[...],
                              preferred_element_type=jnp.float32) + be1a_ref[...]
    h = jnp.maximum(h1, 0.0)
    eh = jnp.dot(h, we1b_ref[...], preferred_element_type=jnp.float32) + be1b_ref[...]
    t = (1.0 + eps_ref[0, 0]) * ea + eh
    t2 = jnp.maximum(jnp.dot(t, we2a_ref[...],
                             preferred_element_type=jnp.float32) + be2a_ref[...], 0.0)
    o_ref[...] = jnp.dot(t2, we2b_ref[...],
                         preferred_element_type=jnp.float32) + be2b_ref[...]


def _node_body(p_ref, x_ref, deg_ref, wn2a_ref, bn2a_ref, wn2b_ref, bn2b_ref,
               wn1a_ref, bn1a_ref, wn1b_ref, bn1b_ref, eps_ref, o_ref):
    agg = p_ref[0] + p_ref[1]
    x = x_ref[...]
    nb = jnp.dot(jnp.maximum(jnp.dot(agg, wn2a_ref[...],
                                     preferred_element_type=jnp.float32)
                             + bn2a_ref[...], 0.0),
                 wn2b_ref[...], preferred_element_type=jnp.float32) + bn2b_ref[...]
    nb = nb - x * deg_ref[...]
    t = (1.0 + eps_ref[0, 0]) * x + nb
    o_ref[...] = jnp.dot(jnp.maximum(jnp.dot(t, wn1a_ref[...],
                                             preferred_element_type=jnp.float32)
                                     + bn1a_ref[...], 0.0),
                         wn1b_ref[...], preferred_element_type=jnp.float32) + bn1b_ref[...]


def _full(shape):
    return pl.BlockSpec(shape, lambda i: tuple(0 for _ in shape))


def kernel(x, edge_index, edge_attr, degree,
           W_e1a, b_e1a, W_e1b, b_e1b,
           W_e2a, b_e2a, W_e2b, b_e2b,
           W_n2a, b_n2a, W_n2b, b_n2b,
           W_n1a, b_n1a, W_n1b, b_n1b,
           eps_node, eps_edge):
    src = edge_index[0]
    dst = edge_index[1]

    xw = pl.pallas_call(
        _xw_body,
        grid=(N_NODES // BN,),
        in_specs=[pl.BlockSpec((BN, D_N), lambda i: (i, 0)),
                  _full((D_N, D_CAT))],
        out_specs=pl.BlockSpec((BN, D_CAT), lambda i: (i, 0)),
        out_shape=jax.ShapeDtypeStruct((N_NODES, D_CAT), jnp.float32),
    )(x, W_e1a[D_E:])

    gath = _gather_kernel(xw, src, dst)

    edge_out = pl.pallas_call(
        _edge_body,
        grid=(N_EDGES // BE,),
        in_specs=[pl.BlockSpec((BE, D_CAT), lambda i: (i, 0)),
                  pl.BlockSpec((BE, D_E), lambda i: (i, 0)),
                  _full((D_E, D_CAT)), _full((1, D_CAT)),
                  _full((D_CAT, D_E)), _full((1, D_E)),
                  _full((D_E, D_E)), _full((1, D_E)),
                  _full((D_E, D_E)), _full((1, D_E)),
                  _full((1, 1))],
        out_specs=pl.BlockSpec((BE, D_E), lambda i: (i, 0)),
        out_shape=jax.ShapeDtypeStruct((N_EDGES, D_E), jnp.float32),
    )(gath, edge_attr, W_e1a[:D_E], b_e1a.reshape(1, D_CAT),
      W_e1b, b_e1b.reshape(1, D_E),
      W_e2a, b_e2a.reshape(1, D_E),
      W_e2b, b_e2b.reshape(1, D_E), eps_edge)

    parts = _scatter_kernel(edge_out, src, dst)

    deg_f = degree.astype(jnp.float32).reshape(N_NODES, 1)
    node_out = pl.pallas_call(
        _node_body,
        grid=(N_NODES // BN,),
        in_specs=[pl.BlockSpec((NC, BN, D_E), lambda i: (0, i, 0)),
                  pl.BlockSpec((BN, D_N), lambda i: (i, 0)),
                  pl.BlockSpec((BN, 1), lambda i: (i, 0)),
                  _full((D_E, D_N)), _full((1, D_N)),
                  _full((D_N, D_N)), _full((1, D_N)),
                  _full((D_N, D_N)), _full((1, D_N)),
                  _full((D_N, D_N)), _full((1, D_N)),
                  _full((1, 1))],
        out_specs=pl.BlockSpec((BN, D_N), lambda i: (i, 0)),
        out_shape=jax.ShapeDtypeStruct((N_NODES, D_N), jnp.float32),
    )(parts, x, deg_f,
      W_n2a, b_n2a.reshape(1, D_N), W_n2b, b_n2b.reshape(1, D_N),
      W_n1a, b_n1a.reshape(1, D_N), W_n1b, b_n1b.reshape(1, D_N),
      eps_node)

    return (node_out, edge_out)


# speedup vs baseline: 2.2868x; 2.2868x over previous
import functools

import jax
import jax.numpy as jnp
from jax import lax
from jax.experimental import pallas as pl
from jax.experimental.pallas import tpu as pltpu
from jax.experimental.pallas import tpu_sc as plsc

N_NODES = 10000
N_EDGES = 320000
D_N = 128
D_E = 16
D_CAT = 144

NC = 2
NS = 16
NW = NC * NS
E_PER_W = N_EDGES // NW
ROWS_PER_TILE = N_NODES // NS

GC = 80
N_GCHUNK = E_PER_W // GC
SCC = 80
N_SCHUNK = E_PER_W // SCC

BE = 6400
BN = 2000

_mesh = plsc.VectorSubcoreMesh(core_axis_name="c", subcore_axis_name="s")
_sc_params = pltpu.CompilerParams(use_tc_tiling_on_sc=False)


@functools.partial(
    pl.kernel,
    out_type=jax.ShapeDtypeStruct((N_EDGES, D_CAT), jnp.float32),
    mesh=_mesh,
    scratch_types=[
        pltpu.VMEM((GC,), jnp.int32),
        pltpu.VMEM((GC,), jnp.int32),
        pltpu.VMEM((GC, D_CAT), jnp.float32),
        pltpu.SemaphoreType.DMA,
    ],
    compiler_params=_sc_params,
)
def _gather_kernel(xw_hbm, src_hbm, dst_hbm, out_hbm, sidx, didx, rows, sem):
    c = lax.axis_index("c")
    s = lax.axis_index("s")
    base = (s * NC + c) * E_PER_W

    def body(k, carry):
        off = base + k * GC
        pltpu.sync_copy(src_hbm.at[pl.ds(off, GC)], sidx)
        pltpu.sync_copy(dst_hbm.at[pl.ds(off, GC)], didx)
        pltpu.async_copy(xw_hbm.at[sidx], rows, sem).wait()
        pltpu.async_copy(xw_hbm.at[didx], rows, sem, add=True).wait()
        pltpu.sync_copy(rows, out_hbm.at[pl.ds(off, GC)])
        return carry

    lax.fori_loop(0, N_GCHUNK, body, 0)


@functools.partial(
    pl.kernel,
    out_type=jax.ShapeDtypeStruct((NC, N_NODES, D_E), jnp.float32),
    mesh=_mesh,
    scratch_types=[
        pltpu.VMEM((SCC,), jnp.int32),
        pltpu.VMEM((SCC,), jnp.int32),
        pltpu.VMEM((SCC, D_E), jnp.float32),
        pltpu.VMEM((ROWS_PER_TILE, D_E), jnp.float32),
        pltpu.VMEM_SHARED((N_NODES, D_E), jnp.float32),
        pltpu.SemaphoreType.DMA,
    ],
    compiler_params=_sc_params,
)
def _scatter_kernel(eo_hbm, src_hbm, dst_hbm, out_hbm,
                    sidx, didx, vals, zbuf, aggsh, sem):
    c = lax.axis_index("c")
    s = lax.axis_index("s")
    wid = s * NC + c

    def zb(i, carry):
        zbuf[i, :] = jnp.zeros((D_E,), jnp.float32)
        return carry

    lax.fori_loop(0, ROWS_PER_TILE, zb, 0)
    pltpu.sync_copy(zbuf, aggsh.at[pl.ds(s * ROWS_PER_TILE, ROWS_PER_TILE)])
    plsc.subcore_barrier()

    def body(k, carry):
        off = wid * E_PER_W + k * SCC
        pltpu.sync_copy(src_hbm.at[pl.ds(off, SCC)], sidx)
        pltpu.sync_copy(dst_hbm.at[pl.ds(off, SCC)], didx)
        pltpu.sync_copy(eo_hbm.at[pl.ds(off, SCC)], vals)
        pltpu.sync_copy(vals, aggsh.at[sidx], add=True)
        pltpu.sync_copy(vals, aggsh.at[didx], add=True)
        return carry

    lax.fori_loop(0, N_SCHUNK, body, 0)
    plsc.subcore_barrier()
    pltpu.sync_copy(
        aggsh.at[pl.ds(s * ROWS_PER_TILE, ROWS_PER_TILE)],
        out_hbm.at[c, pl.ds(s * ROWS_PER_TILE, ROWS_PER_TILE)],
    )


def _xw_body(x_ref, w_ref, o_ref):
    o_ref[...] = jnp.dot(x_ref[...], w_ref[...],
                         preferred_element_type=jnp.float32)


def _edge_body(g_ref, ea_ref, wtop_ref, be1a_ref, we1b_ref, be1b_ref,
               we2a_ref, be2a_ref, we2b_ref, be2b_ref, eps_ref, o_ref):
    ea = ea_ref[...]
    h1 = g_ref[...] + jnp.dot(ea, wtop_ref[...],
                              preferred_element_type=jnp.float32) + be1a_ref[...]
    h = jnp.maximum(h1, 0.0)
    eh = jnp.dot(h, we1b_ref[...], preferred_element_type=jnp.float32) + be1b_ref[...]
    t = (1.0 + eps_ref[0, 0]) * ea + eh
    t2 = jnp.maximum(jnp.dot(t, we2a_ref[...],
                             preferred_element_type=jnp.float32) + be2a_ref[...], 0.0)
    o_ref[...] = jnp.dot(t2, we2b_ref[...],
                         preferred_element_type=jnp.float32) + be2b_ref[...]


def _node_body(p_ref, x_ref, deg_ref, wn2a_ref, bn2a_ref, wn2b_ref, bn2b_ref,
               wn1a_ref, bn1a_ref, wn1b_ref, bn1b_ref, eps_ref, o_ref):
    agg = p_ref[0] + p_ref[1]
    x = x_ref[...]
    nb = jnp.dot(jnp.maximum(jnp.dot(agg, wn2a_ref[...],
                                     preferred_element_type=jnp.float32)
                             + bn2a_ref[...], 0.0),
                 wn2b_ref[...], preferred_element_type=jnp.float32) + bn2b_ref[...]
    nb = nb - x * deg_ref[...]
    t = (1.0 + eps_ref[0, 0]) * x + nb
    o_ref[...] = jnp.dot(jnp.maximum(jnp.dot(t, wn1a_ref[...],
                                             preferred_element_type=jnp.float32)
                                     + bn1a_ref[...], 0.0),
                         wn1b_ref[...], preferred_element_type=jnp.float32) + bn1b_ref[...]


def _full(shape):
    return pl.BlockSpec(shape, lambda i: tuple(0 for _ in shape))


def kernel(x, edge_index, edge_attr, degree,
           W_e1a, b_e1a, W_e1b, b_e1b,
           W_e2a, b_e2a, W_e2b, b_e2b,
           W_n2a, b_n2a, W_n2b, b_n2b,
           W_n1a, b_n1a, W_n1b, b_n1b,
           eps_node, eps_edge):
    src = edge_index[0]
    dst = edge_index[1]

    xw = pl.pallas_call(
        _xw_body,
        grid=(N_NODES // BN,),
        in_specs=[pl.BlockSpec((BN, D_N), lambda i: (i, 0)),
                  _full((D_N, D_CAT))],
        out_specs=pl.BlockSpec((BN, D_CAT), lambda i: (i, 0)),
        out_shape=jax.ShapeDtypeStruct((N_NODES, D_CAT), jnp.float32),
    )(x, W_e1a[D_E:])

    gath = _gather_kernel(xw, src, dst)

    edge_out = pl.pallas_call(
        _edge_body,
        grid=(N_EDGES // BE,),
        in_specs=[pl.BlockSpec((BE, D_CAT), lambda i: (i, 0)),
                  pl.BlockSpec((BE, D_E), lambda i: (i, 0)),
                  _full((D_E, D_CAT)), _full((1, D_CAT)),
                  _full((D_CAT, D_E)), _full((1, D_E)),
                  _full((D_E, D_E)), _full((1, D_E)),
                  _full((D_E, D_E)), _full((1, D_E)),
                  _full((1, 1))],
        out_specs=pl.BlockSpec((BE, D_E), lambda i: (i, 0)),
        out_shape=jax.ShapeDtypeStruct((N_EDGES, D_E), jnp.float32),
    )(gath, edge_attr, W_e1a[:D_E], b_e1a.reshape(1, D_CAT),
      W_e1b, b_e1b.reshape(1, D_E),
      W_e2a, b_e2a.reshape(1, D_E),
      W_e2b, b_e2b.reshape(1, D_E), eps_edge)

    parts = _scatter_kernel(edge_out, src, dst)

    deg_f = degree.astype(jnp.float32).reshape(N_NODES, 1)
    node_out = pl.pallas_call(
        _node_body,
        grid=(N_NODES // BN,),
        in_specs=[pl.BlockSpec((NC, BN, D_E), lambda i: (0, i, 0)),
                  pl.BlockSpec((BN, D_N), lambda i: (i, 0)),
                  pl.BlockSpec((BN, 1), lambda i: (i, 0)),
                  _full((D_E, D_N)), _full((1, D_N)),
                  _full((D_N, D_N)), _full((1, D_N)),
                  _full((D_N, D_N)), _full((1, D_N)),
                  _full((D_N, D_N)), _full((1, D_N)),
                  _full((1, 1))],
        out_specs=pl.BlockSpec((BN, D_N), lambda i: (i, 0)),
        out_shape=jax.ShapeDtypeStruct((N_NODES, D_N), jnp.float32),
    )(parts, x, deg_f,
      W_n2a, b_n2a.reshape(1, D_N), W_n2b, b_n2b.reshape(1, D_N),
      W_n1a, b_n1a.reshape(1, D_N), W_n1b, b_n1b.reshape(1, D_N),
      eps_node)

    return (node_out, edge_out)


# speedup vs baseline: 2.8410x; 1.2423x over previous
import functools

import jax
import jax.numpy as jnp
from jax import lax
from jax.experimental import pallas as pl
from jax.experimental.pallas import tpu as pltpu
from jax.experimental.pallas import tpu_sc as plsc

N_NODES = 10000
N_EDGES = 320000
D_N = 128
D_E = 16
D_CAT = 144

NC = 2
NS = 16
NW = NC * NS
E_PER_W = N_EDGES // NW
ROWS_PER_TILE = N_NODES // NS

GC = 100
NCH = E_PER_W // GC
NBUF = 4
SUPC = 20
NSUP = NCH // SUPC

BE = 6400
BN = 2000

_mesh = plsc.VectorSubcoreMesh(core_axis_name="c", subcore_axis_name="s")
_sc_params = pltpu.CompilerParams(use_tc_tiling_on_sc=False)


@functools.partial(
    pl.kernel,
    out_type=jax.ShapeDtypeStruct((N_EDGES, D_CAT), jnp.float32),
    mesh=_mesh,
    scratch_types=[
        pltpu.VMEM((NCH, GC), jnp.int32),
        pltpu.VMEM((NCH, GC), jnp.int32),
        [pltpu.VMEM((GC, D_CAT), jnp.float32)] * NBUF,
        [pltpu.SemaphoreType.DMA] * NBUF,
        [pltpu.SemaphoreType.DMA] * NBUF,
    ],
    compiler_params=_sc_params,
)
def _gather_kernel(xw_hbm, src3_hbm, dst3_hbm, out_hbm,
                   sidx, didx, bufs, semg, semw):
    c = lax.axis_index("c")
    s = lax.axis_index("s")
    wid = s * NC + c
    base = wid * E_PER_W

    pltpu.sync_copy(src3_hbm.at[wid], sidx)
    pltpu.sync_copy(dst3_hbm.at[wid], didx)

    def fire_s(k, b):
        pltpu.async_copy(xw_hbm.at[sidx.at[k]], bufs[b], semg[b])

    def fire_d(k, b):
        pltpu.async_copy(xw_hbm.at[didx.at[k]], bufs[b], semg[b], add=True)

    def wait_g(b):
        pltpu.make_async_copy(xw_hbm.at[sidx.at[0]], bufs[b], semg[b]).wait()

    def fire_w(k, b):
        pltpu.async_copy(bufs[b], out_hbm.at[pl.ds(base + k * GC, GC)], semw[b])

    def wait_w(b):
        pltpu.make_async_copy(
            bufs[b], out_hbm.at[pl.ds(base, GC)], semw[b]).wait()

    fire_s(0, 0)

    def group(g, carry):
        for b in range(NBUF):
            k = g * NBUF + b
            bn = (b + 1) % NBUF

            @pl.when(jnp.logical_and(k >= NBUF - 1, k + 1 < NCH))
            def _():
                wait_w(bn)

            @pl.when(k + 1 < NCH)
            def _():
                fire_s(k + 1, bn)

            wait_g(b)
            fire_d(k, b)
            wait_g(b)
            fire_w(k, b)
        return carry

    lax.fori_loop(0, NCH // NBUF, group, 0)
    for b in range(NBUF):
        wait_w(b)


@functools.partial(
    pl.kernel,
    out_type=jax.ShapeDtypeStruct((NC, N_NODES, D_E), jnp.float32),
    mesh=_mesh,
    scratch_types=[
        pltpu.VMEM((NCH, GC), jnp.int32),
        pltpu.VMEM((NCH, GC), jnp.int32),
        [pltpu.VMEM((SUPC, GC, D_E), jnp.float32)] * 2,
        pltpu.VMEM((ROWS_PER_TILE, D_E), jnp.float32),
        pltpu.VMEM_SHARED((N_NODES, D_E), jnp.float32),
        [pltpu.SemaphoreType.DMA] * 2,
        pltpu.SemaphoreType.DMA,
    ],
    compiler_params=_sc_params,
)
def _scatter_kernel(eo4_hbm, src3_hbm, dst3_hbm, out_hbm,
                    sidx, didx, vbufs, zbuf, aggsh, semv, sems):
    c = lax.axis_index("c")
    s = lax.axis_index("s")
    wid = s * NC + c

    def zb(i, carry):
        zbuf[i, :] = jnp.zeros((D_E,), jnp.float32)
        return carry

    lax.fori_loop(0, ROWS_PER_TILE, zb, 0)
    pltpu.sync_copy(zbuf, aggsh.at[pl.ds(s * ROWS_PER_TILE, ROWS_PER_TILE)])
    plsc.subcore_barrier()

    pltpu.sync_copy(src3_hbm.at[wid], sidx)
    pltpu.sync_copy(dst3_hbm.at[wid], didx)
    pltpu.async_copy(eo4_hbm.at[wid, 0], vbufs[0], semv[0])

    def drain_scat(v):
        def w(j, carry):
            pltpu.make_async_copy(
                v.at[0], aggsh.at[sidx.at[0]], sems).wait()
            return carry
        lax.fori_loop(0, 2 * SUPC, w, 0)

    for sp in range(NSUP):
        bv = sp % 2
        pltpu.make_async_copy(eo4_hbm.at[wid, sp], vbufs[bv], semv[bv]).wait()
        if sp >= 1:
            drain_scat(vbufs[1 - bv])
        if sp + 1 < NSUP:
            pltpu.async_copy(eo4_hbm.at[wid, sp + 1],
                             vbufs[1 - bv], semv[1 - bv])

        def fire(j, carry):
            k = sp * SUPC + j
            pltpu.async_copy(vbufs[bv].at[j], aggsh.at[sidx.at[k]],
                             sems, add=True)
            pltpu.async_copy(vbufs[bv].at[j], aggsh.at[didx.at[k]],
                             sems, add=True)
            return carry

        lax.fori_loop(0, SUPC, fire, 0)

    drain_scat(vbufs[(NSUP - 1) % 2])
    plsc.subcore_barrier()
    pltpu.sync_copy(
        aggsh.at[pl.ds(s * ROWS_PER_TILE, ROWS_PER_TILE)],
        out_hbm.at[c, pl.ds(s * ROWS_PER_TILE, ROWS_PER_TILE)],
    )


def _xw_body(x_ref, w_ref, o_ref):
    o_ref[...] = jnp.dot(x_ref[...], w_ref[...],
                         preferred_element_type=jnp.float32)


def _edge_body(g_ref, ea_ref, wtop_ref, be1a_ref, we1b_ref, be1b_ref,
               we2a_ref, be2a_ref, we2b_ref, be2b_ref, eps_ref, o_ref):
    ea = ea_ref[...]
    h1 = g_ref[...] + jnp.dot(ea, wtop_ref[...],
                              preferred_element_type=jnp.float32) + be1a_ref[...]
    h = jnp.maximum(h1, 0.0)
    eh = jnp.dot(h, we1b_ref[...], preferred_element_type=jnp.float32) + be1b_ref[...]
    t = (1.0 + eps_ref[0, 0]) * ea + eh
    t2 = jnp.maximum(jnp.dot(t, we2a_ref[...],
                             preferred_element_type=jnp.float32) + be2a_ref[...], 0.0)
    o_ref[...] = jnp.dot(t2, we2b_ref[...],
                         preferred_element_type=jnp.float32) + be2b_ref[...]


def _node_body(p_ref, x_ref, deg_ref, wn2a_ref, bn2a_ref, wn2b_ref, bn2b_ref,
               wn1a_ref, bn1a_ref, wn1b_ref, bn1b_ref, eps_ref, o_ref):
    agg = p_ref[0] + p_ref[1]
    x = x_ref[...]
    nb = jnp.dot(jnp.maximum(jnp.dot(agg, wn2a_ref[...],
                                     preferred_element_type=jnp.float32)
                             + bn2a_ref[...], 0.0),
                 wn2b_ref[...], preferred_element_type=jnp.float32) + bn2b_ref[...]
    nb = nb - x * deg_ref[...]
    t = (1.0 + eps_ref[0, 0]) * x + nb
    o_ref[...] = jnp.dot(jnp.maximum(jnp.dot(t, wn1a_ref[...],
                                             preferred_element_type=jnp.float32)
                                     + bn1a_ref[...], 0.0),
                         wn1b_ref[...], preferred_element_type=jnp.float32) + bn1b_ref[...]


def _full(shape):
    return pl.BlockSpec(shape, lambda i: tuple(0 for _ in shape))


def kernel(x, edge_index, edge_attr, degree,
           W_e1a, b_e1a, W_e1b, b_e1b,
           W_e2a, b_e2a, W_e2b, b_e2b,
           W_n2a, b_n2a, W_n2b, b_n2b,
           W_n1a, b_n1a, W_n1b, b_n1b,
           eps_node, eps_edge):
    src = edge_index[0]
    dst = edge_index[1]
    src3 = src.reshape(NW, NCH, GC)
    dst3 = dst.reshape(NW, NCH, GC)

    xw = pl.pallas_call(
        _xw_body,
        grid=(N_NODES // BN,),
        in_specs=[pl.BlockSpec((BN, D_N), lambda i: (i, 0)),
                  _full((D_N, D_CAT))],
        out_specs=pl.BlockSpec((BN, D_CAT), lambda i: (i, 0)),
        out_shape=jax.ShapeDtypeStruct((N_NODES, D_CAT), jnp.float32),
    )(x, W_e1a[D_E:])

    gath = _gather_kernel(xw, src3, dst3)

    edge_out = pl.pallas_call(
        _edge_body,
        grid=(N_EDGES // BE,),
        in_specs=[pl.BlockSpec((BE, D_CAT), lambda i: (i, 0)),
                  pl.BlockSpec((BE, D_E), lambda i: (i, 0)),
                  _full((D_E, D_CAT)), _full((1, D_CAT)),
                  _full((D_CAT, D_E)), _full((1, D_E)),
                  _full((D_E, D_E)), _full((1, D_E)),
                  _full((D_E, D_E)), _full((1, D_E)),
                  _full((1, 1))],
        out_specs=pl.BlockSpec((BE, D_E), lambda i: (i, 0)),
        out_shape=jax.ShapeDtypeStruct((N_EDGES, D_E), jnp.float32),
    )(gath, edge_attr, W_e1a[:D_E], b_e1a.reshape(1, D_CAT),
      W_e1b, b_e1b.reshape(1, D_E),
      W_e2a, b_e2a.reshape(1, D_E),
      W_e2b, b_e2b.reshape(1, D_E), eps_edge)

    eo4 = edge_out.reshape(NW, NSUP, SUPC, GC, D_E)
    parts = _scatter_kernel(eo4, src3, dst3)

    deg_f = degree.astype(jnp.float32).reshape(N_NODES, 1)
    node_out = pl.pallas_call(
        _node_body,
        grid=(N_NODES // BN,),
        in_specs=[pl.BlockSpec((NC, BN, D_E), lambda i: (0, i, 0)),
                  pl.BlockSpec((BN, D_N), lambda i: (i, 0)),
                  pl.BlockSpec((BN, 1), lambda i: (i, 0)),
                  _full((D_E, D_N)), _full((1, D_N)),
                  _full((D_N, D_N)), _full((1, D_N)),
                  _full((D_N, D_N)), _full((1, D_N)),
                  _full((D_N, D_N)), _full((1, D_N)),
                  _full((1, 1))],
        out_specs=pl.BlockSpec((BN, D_N), lambda i: (i, 0)),
        out_shape=jax.ShapeDtypeStruct((N_NODES, D_N), jnp.float32),
    )(parts, x, deg_f,
      W_n2a, b_n2a.reshape(1, D_N), W_n2b, b_n2b.reshape(1, D_N),
      W_n1a, b_n1a.reshape(1, D_N), W_n1b, b_n1b.reshape(1, D_N),
      eps_node)

    return (node_out, edge_out)
